# Initial kernel scaffold; baseline (speedup 1.0000x reference)
import jax
import jax.numpy as jnp
from jax.experimental import pallas as pl


def kernel(x, params, edge_index, batch):
    raise NotImplementedError("write your pallas kernel here")


# speedup vs baseline: 5.0903x; 5.0903x over previous
import functools

import jax
import jax.numpy as jnp
from jax import lax
from jax.experimental import pallas as pl
from jax.experimental.pallas import tpu as pltpu
from jax.experimental.pallas import tpu_sc as plsc

B64 = 64
BN = 2048
WIN = 1024
CH = 128
NCH = WIN // CH

_HIGH = jax.lax.Precision.HIGHEST


def _mm(a, b):
    return jax.lax.dot_general(a, b, (((1,), (0,)), ((), ())),
                               precision=_HIGH,
                               preferred_element_type=jnp.float32)


def _mmT(a, b):
    return jax.lax.dot_general(a, b, (((0,), (0,)), ((), ())),
                               precision=_HIGH,
                               preferred_element_type=jnp.float32)


def _onehot(batch_blk):
    g = jax.lax.broadcasted_iota(jnp.int32, (1, B64), 1)
    return (batch_blk == g).astype(jnp.float32)


def _sc_agg(tab, srcr, dstr, blocks, split, npad):
    erows = srcr.shape[0]
    nb = len(blocks)
    nout = 2 if split else nb
    rpt = npad // 16
    zr = rpt // 32
    if split:
        nw = erows // (32 * NCH)
    else:
        nw = erows // (16 * NCH)

    mesh = plsc.VectorSubcoreMesh(core_axis_name="c", subcore_axis_name="s",
                                  num_cores=2, num_subcores=16)

    def body(tab_r, src_r, dst_r, out_r, srcv, dstv, idxv, rowsv, zv, acc, sem):
        c = lax.axis_index("c")
        s = lax.axis_index("s")

        def zfill(i, _):
            zv[i, :] = jnp.zeros((16,), jnp.float32)
            return 0
        lax.fori_loop(0, zr, zfill, 0)

        def window(base_row, mult, off, raw_idx):
            def win(w, _):
                ro = base_row + w * NCH
                pltpu.sync_copy(src_r.at[pl.ds(ro, NCH)], srcv)
                pltpu.sync_copy(dst_r.at[pl.ds(ro, NCH)], dstv)
                if not raw_idx:
                    def idxbody(j, _):
                        r = j // 8
                        cc = (j % 8) * 16
                        v = srcv[r, pl.ds(cc, 16)]
                        idxv[r, pl.ds(cc, 16)] = v * mult + off
                        return 0
                    lax.fori_loop(0, NCH * 8, idxbody, 0)
                    iref = idxv
                else:
                    iref = srcv
                descs = [
                    pltpu.async_copy(tab_r.at[iref.at[j]],
                                     rowsv.at[pl.ds(j * CH, CH)], sem)
                    for j in range(NCH)
                ]
                for d in descs:
                    d.wait()
                for j in range(NCH):
                    pltpu.sync_copy(rowsv.at[pl.ds(j * CH, CH)],
                                    acc.at[dstv.at[j]], add=True)
                return 0
            lax.fori_loop(0, nw, win, 0)

        def zero_acc():
            for k in range(32):
                pltpu.sync_copy(zv, acc.at[pl.ds(s * rpt + k * zr, zr)])

        def writeout(obi):
            pltpu.sync_copy(acc.at[pl.ds(s * rpt, rpt)],
                            out_r.at[obi, pl.ds(s * rpt, rpt), :])

        if split:
            mult, off = blocks[0]
            zero_acc()
            plsc.subcore_barrier()
            base = (c * 16 + s) * nw * NCH
            window(base, mult, off, raw_idx=(mult == 1 and off == 0))
            plsc.subcore_barrier()
            writeout(c)
            plsc.subcore_barrier()
        else:
            nrounds = (nb + 1) // 2
            for r in range(nrounds):
                f0, f1 = 2 * r, 2 * r + 1
                if f1 < nb:
                    mult = jnp.where(c == 0, blocks[f0][0], blocks[f1][0])
                    off = jnp.where(c == 0, blocks[f0][1], blocks[f1][1])
                    active = None
                else:
                    mult = jnp.int32(blocks[f0][0])
                    off = jnp.int32(blocks[f0][1])
                    active = (c == 0)
                zero_acc()
                plsc.subcore_barrier()
                base = s * nw * NCH
                if active is None:
                    window(base, mult, off, raw_idx=False)
                else:
                    @pl.when(active)
                    def _():
                        window(base, mult, off, raw_idx=False)
                plsc.subcore_barrier()
                obi = f0 + c
                if active is None:
                    writeout(obi)
                else:
                    @pl.when(active)
                    def _():
                        writeout(f0)
                plsc.subcore_barrier()

    kern = pl.kernel(
        body,
        out_type=jax.ShapeDtypeStruct((nout, npad, 16), jnp.float32),
        mesh=mesh,
        scratch_types=[
            pltpu.VMEM((NCH, CH), jnp.int32),
            pltpu.VMEM((NCH, CH), jnp.int32),
            pltpu.VMEM((NCH, CH), jnp.int32),
            pltpu.VMEM((WIN, 16), jnp.float32),
            pltpu.VMEM((zr, 16), jnp.float32),
            pltpu.VMEM_SHARED((npad, 16), jnp.float32),
            pltpu.SemaphoreType.DMA,
        ],
        compiler_params=pltpu.CompilerParams(use_tc_tiling_on_sc=False),
    )
    return kern(tab, srcr, dstr)


def _k_pre(xp, bt, w0, npad):
    grid = npad // BN

    def body(x_r, b_r, w_r, p0_r, cg_r):
        i = pl.program_id(0)
        mk = (b_r[...] < B64).astype(jnp.float32)
        p0_r[...] = x_r[...] * w_r[0:1, :]
        oh = _onehot(b_r[...])

        @pl.when(i == 0)
        def _():
            cg_r[...] = jnp.zeros_like(cg_r)
        cg_r[...] += _mmT(oh, mk)

    return pl.pallas_call(
        body,
        grid=(grid,),
        in_specs=[
            pl.BlockSpec((BN, 1), lambda i: (i, 0)),
            pl.BlockSpec((BN, 1), lambda i: (i, 0)),
            pl.BlockSpec((8, 16), lambda i: (0, 0)),
        ],
        out_specs=[
            pl.BlockSpec((BN, 16), lambda i: (i, 0)),
            pl.BlockSpec((B64, 1), lambda i: (0, 0)),
        ],
        out_shape=[
            jax.ShapeDtypeStruct((npad, 16), jnp.float32),
            jax.ShapeDtypeStruct((B64, 1), jnp.float32),
        ],
    )(xp, bt, w0)


def _k_inv(agg0, npad):
    grid = npad // BN

    def body(c_r, inv_r):
        deg = c_r[0][:, 0:1]
        inv_r[...] = 1.0 / jnp.maximum(deg, 1.0)

    return pl.pallas_call(
        body,
        grid=(grid,),
        in_specs=[pl.BlockSpec((1, BN, 16), lambda i: (1, i, 0))],
        out_specs=pl.BlockSpec((BN, 1), lambda i: (i, 0)),
        out_shape=jax.ShapeDtypeStruct((npad, 1), jnp.float32),
    )(agg0)


def _k1(agg, h, inv, bt, wl, wr, cp, mode, wout, npad):
    grid = npad // BN
    a_dim = agg.shape[0]
    hw = h.shape[1]

    def body(a_r, h_r, inv_r, b_r, wl_r, wr_r, cp_r, z_r, s1_r):
        i = pl.program_id(0)
        inv = inv_r[...]
        if mode == "l0":
            acc = a_r[0] * inv
            rterm = h_r[...] * wr_r[0:1, :]
        elif mode == "sum2":
            acc = _mm((a_r[0] + a_r[1]) * inv, wl_r[...])
            rterm = _mm(h_r[...], wr_r[...])
        else:
            acc = _mm(a_r[0] * inv, wl_r[pl.ds(0, 16), :])
            for f in range(1, 7):
                acc += _mm(a_r[f] * inv, wl_r[pl.ds(f * 16, 16), :])
            rterm = _mm(h_r[...], wr_r[...])
        mk = (b_r[...] < B64).astype(jnp.float32)
        z = (acc + rterm + cp_r[3:4, :]) * mk
        z_r[...] = z
        oh = _onehot(b_r[...])

        @pl.when(i == 0)
        def _():
            s1_r[...] = jnp.zeros_like(s1_r)
        s1_r[...] += _mmT(oh, z)

    return pl.pallas_call(
        body,
        grid=(grid,),
        in_specs=[
            pl.BlockSpec((a_dim, BN, 16), lambda i: (0, i, 0)),
            pl.BlockSpec((BN, hw), lambda i: (i, 0)),
            pl.BlockSpec((BN, 1), lambda i: (i, 0)),
            pl.BlockSpec((BN, 1), lambda i: (i, 0)),
            pl.BlockSpec(wl.shape, lambda i: (0, 0)),
            pl.BlockSpec(wr.shape, lambda i: (0, 0)),
            pl.BlockSpec((8, wout), lambda i: (0, 0)),
        ],
        out_specs=[
            pl.BlockSpec((BN, wout), lambda i: (i, 0)),
            pl.BlockSpec((B64, wout), lambda i: (0, 0)),
        ],
        out_shape=[
            jax.ShapeDtypeStruct((npad, wout), jnp.float32),
            jax.ShapeDtypeStruct((B64, wout), jnp.float32),
        ],
    )(agg, h, inv, bt, wl, wr, cp)


def _k2(z, s1, cg, bt, cp, wout, npad):
    grid = npad // BN

    def body(z_r, s1_r, cg_r, b_r, cp_r, s2_r):
        i = pl.program_id(0)
        mean_g = s1_r[...] / jnp.maximum(cg_r[...], 1.0)
        oh = _onehot(b_r[...])
        meanb = _mm(oh, mean_g)
        clone = z_r[...] - cp_r[0:1, :] * meanb

        @pl.when(i == 0)
        def _():
            s2_r[...] = jnp.zeros_like(s2_r)
        s2_r[...] += _mmT(oh, clone * clone)

    return pl.pallas_call(
        body,
        grid=(grid,),
        in_specs=[
            pl.BlockSpec((BN, wout), lambda i: (i, 0)),
            pl.BlockSpec((B64, wout), lambda i: (0, 0)),
            pl.BlockSpec((B64, 1), lambda i: (0, 0)),
            pl.BlockSpec((BN, 1), lambda i: (i, 0)),
            pl.BlockSpec((8, wout), lambda i: (0, 0)),
        ],
        out_specs=pl.BlockSpec((B64, wout), lambda i: (0, 0)),
        out_shape=jax.ShapeDtypeStruct((B64, wout), jnp.float32),
    )(z, s1, cg, bt, cp)


def _k3(z, s1, s2, cg, bt, cp, wout, npad):
    grid = npad // BN

    def body(z_r, s1_r, s2_r, cg_r, b_r, cp_r, h_r):
        cg = cg_r[...]
        mean_g = s1_r[...] / jnp.maximum(cg, 1.0)
        sq = jnp.floor(jnp.sqrt(cg))
        s2 = s2_r[...]
        factor = jnp.where(s2 > 0.0, sq / jnp.sqrt(s2), 0.0)
        oh = _onehot(b_r[...])
        meanb = _mm(oh, mean_g)
        fb = _mm(oh, factor)
        clone = z_r[...] - cp_r[0:1, :] * meanb
        mk = (b_r[...] < B64).astype(jnp.float32)
        h_r[...] = jnp.maximum(clone * fb * cp_r[1:2, :] + cp_r[2:3, :],
                               0.0) * mk

    return pl.pallas_call(
        body,
        grid=(grid,),
        in_specs=[
            pl.BlockSpec((BN, wout), lambda i: (i, 0)),
            pl.BlockSpec((B64, wout), lambda i: (0, 0)),
            pl.BlockSpec((B64, wout), lambda i: (0, 0)),
            pl.BlockSpec((B64, 1), lambda i: (0, 0)),
            pl.BlockSpec((BN, 1), lambda i: (i, 0)),
            pl.BlockSpec((8, wout), lambda i: (0, 0)),
        ],
        out_specs=pl.BlockSpec((BN, wout), lambda i: (i, 0)),
        out_shape=jax.ShapeDtypeStruct((npad, wout), jnp.float32),
    )(z, s1, s2, cg, bt, cp)


def _k_yr(h6, w7, npad):
    grid = npad // BN

    def body(h_r, w_r, y_r, r_r):
        yr = _mm(h_r[...], w_r[...])
        y = yr[:, 0:1]
        y_r[...] = jnp.concatenate(
            [y, jnp.zeros((BN, 15), jnp.float32)], axis=1)
        r_r[...] = yr[:, 1:2]

    return pl.pallas_call(
        body,
        grid=(grid,),
        in_specs=[
            pl.BlockSpec((BN, 112), lambda i: (i, 0)),
            pl.BlockSpec((112, 8), lambda i: (0, 0)),
        ],
        out_specs=[
            pl.BlockSpec((BN, 16), lambda i: (i, 0)),
            pl.BlockSpec((BN, 1), lambda i: (i, 0)),
        ],
        out_shape=[
            jax.ShapeDtypeStruct((npad, 16), jnp.float32),
            jax.ShapeDtypeStruct((npad, 1), jnp.float32),
        ],
    )(h6, w7)


def _k_final(agg7, inv, r, bt, cg, cp7, npad):
    grid = npad // BN

    def body(a_r, inv_r, r_r, b_r, cg_r, cp_r, ret_r):
        i = pl.program_id(0)
        aggy = (a_r[0] + a_r[1])[:, 0:1]
        mk = (b_r[...] < B64).astype(jnp.float32)
        o = (aggy * inv_r[...] + cp_r[3:4, 0:1] + r_r[...]) * mk
        oh = _onehot(b_r[...])

        @pl.when(i == 0)
        def _():
            ret_r[...] = jnp.zeros_like(ret_r)
        ret_r[...] += _mmT(oh, o)

        @pl.when(i == grid - 1)
        def _():
            ret_r[...] = ret_r[...] / jnp.maximum(cg_r[...], 1.0)

    return pl.pallas_call(
        body,
        grid=(grid,),
        in_specs=[
            pl.BlockSpec((2, BN, 16), lambda i: (0, i, 0)),
            pl.BlockSpec((BN, 1), lambda i: (i, 0)),
            pl.BlockSpec((BN, 1), lambda i: (i, 0)),
            pl.BlockSpec((BN, 1), lambda i: (i, 0)),
            pl.BlockSpec((B64, 1), lambda i: (0, 0)),
            pl.BlockSpec((8, 16), lambda i: (0, 0)),
        ],
        out_specs=pl.BlockSpec((B64, 1), lambda i: (0, 0)),
        out_shape=jax.ShapeDtypeStruct((B64, 1), jnp.float32),
    )(agg7, inv, r, bt, cg, cp7)


def _padT(w, rows, cols):
    wt = w.T.astype(jnp.float32)
    return jnp.pad(wt, ((0, rows - wt.shape[0]), (0, cols - wt.shape[1])))


def kernel(x, params, edge_index, batch):
    N = x.shape[0]
    E = edge_index.shape[1]
    npad = ((N + BN - 1) // BN) * BN
    per_tile = ((E + 32 * WIN - 1) // (32 * WIN)) * WIN
    epad = 32 * per_tile
    erows = epad // CH
    pade = epad - E

    src = edge_index[0].astype(jnp.int32)
    dst = edge_index[1].astype(jnp.int32)
    pidx = jnp.arange(pade, dtype=jnp.int32)
    srcr = jnp.concatenate([src, (pidx * 97) % N]).reshape(erows, CH)
    dstr = jnp.concatenate([dst, N + pidx % (npad - N)]).reshape(erows, CH)

    xp = jnp.pad(x.astype(jnp.float32), ((0, npad - N), (0, 0)))
    bt = jnp.pad(batch.astype(jnp.int32), (0, npad - N),
                 constant_values=B64).reshape(npad, 1)
    ones16 = jnp.ones((npad, 16), jnp.float32)

    L = params["layers"]
    Nm = params["norms"]
    wouts = [16] + [112] * 6
    wl = [None] * 8
    wr = [None] * 8
    cp = [None] * 8
    wl[0] = _padT(L[0]["Wl"], 8, 16)
    wr[0] = _padT(L[0]["Wr"], 8, 16)
    wl[1] = _padT(L[1]["Wl"], 16, 112)
    wr[1] = _padT(L[1]["Wr"], 16, 112)
    for i in range(2, 7):
        wl[i] = _padT(L[i]["Wl"], 112, 112)
        wr[i] = _padT(L[i]["Wr"], 112, 112)
    w7 = jnp.concatenate([_padT(L[7]["Wl"], 112, 1),
                          _padT(L[7]["Wr"], 112, 1),
                          jnp.zeros((112, 6), jnp.float32)], axis=1)
    for i in range(7):
        w = wouts[i]
        cp[i] = jnp.stack([
            jnp.pad(Nm[i]["alpha"].astype(jnp.float32), (0, w - Nm[i]["alpha"].shape[0])),
            jnp.pad(Nm[i]["scale"].astype(jnp.float32), (0, w - Nm[i]["scale"].shape[0])),
            jnp.pad(Nm[i]["shift"].astype(jnp.float32), (0, w - Nm[i]["shift"].shape[0])),
            jnp.pad(L[i]["bl"].astype(jnp.float32), (0, w - L[i]["bl"].shape[0])),
        ] + [jnp.zeros((w,), jnp.float32)] * 4)
    cp7 = jnp.zeros((8, 16), jnp.float32).at[3, 0].set(L[7]["bl"][0])

    p0, cg = _k_pre(xp, bt, wl[0], npad)
    tab0 = jnp.concatenate([p0, ones16], axis=0)
    agg0 = _sc_agg(tab0, srcr, dstr, ((1, 0), (1, npad)), False, npad)
    inv = _k_inv(agg0, npad)
    z, s1 = _k1(agg0, xp, inv, bt, wl[0], wr[0], cp[0], "l0", 16, npad)
    s2 = _k2(z, s1, cg, bt, cp[0], 16, npad)
    h = _k3(z, s1, s2, cg, bt, cp[0], 16, npad)

    agg = _sc_agg(h, srcr, dstr, ((1, 0),), True, npad)
    z, s1 = _k1(agg, h, inv, bt, wl[1], wr[1], cp[1], "sum2", 112, npad)
    s2 = _k2(z, s1, cg, bt, cp[1], 112, npad)
    h = _k3(z, s1, s2, cg, bt, cp[1], 112, npad)

    blocks7 = tuple((7, f) for f in range(7))
    for i in range(2, 7):
        hv = h.reshape(npad * 7, 16)
        agg = _sc_agg(hv, srcr, dstr, blocks7, False, npad)
        z, s1 = _k1(agg, h, inv, bt, wl[i], wr[i], cp[i], "cat7", 112, npad)
        s2 = _k2(z, s1, cg, bt, cp[i], 112, npad)
        h = _k3(z, s1, s2, cg, bt, cp[i], 112, npad)

    y16, r = _k_yr(h, w7, npad)
    agg7 = _sc_agg(y16, srcr, dstr, ((1, 0),), True, npad)
    ret = _k_final(agg7, inv, r, bt, cg, cp7, npad)
    return ret.reshape(B64)


# speedup vs baseline: 5.3998x; 1.0608x over previous
import functools

import jax
import jax.numpy as jnp
from jax import lax
from jax.experimental import pallas as pl
from jax.experimental.pallas import tpu as pltpu
from jax.experimental.pallas import tpu_sc as plsc

B64 = 64
BN = 2048
WIN = 1024
CH = 128
NCH = WIN // CH

_HIGH = jax.lax.Precision.HIGHEST


def _mm(a, b):
    return jax.lax.dot_general(a, b, (((1,), (0,)), ((), ())),
                               precision=_HIGH,
                               preferred_element_type=jnp.float32)


def _mmT(a, b):
    return jax.lax.dot_general(a, b, (((0,), (0,)), ((), ())),
                               precision=_HIGH,
                               preferred_element_type=jnp.float32)


def _onehot(batch_blk):
    g = jax.lax.broadcasted_iota(jnp.int32, (1, B64), 1)
    return (batch_blk == g).astype(jnp.float32)


def _sc_agg(tab, srcr, dstr, blocks, split, npad):
    erows = srcr.shape[0]
    nb = len(blocks)
    odd = (not split) and (nb % 2 == 1) and nb > 1
    nout = 2 if split else nb + (1 if odd else 0)
    rpt = npad // 16
    zr = rpt // 32
    if split:
        nw = erows // (32 * NCH)
    else:
        nw = erows // (16 * NCH)
        nws = nw // 2

    mesh = plsc.VectorSubcoreMesh(core_axis_name="c", subcore_axis_name="s",
                                  num_cores=2, num_subcores=16)

    def body(tab_r, src_r, dst_r, out_r, srcv, dstv, idxv, rowsv, zv, acc, sem):
        c = lax.axis_index("c")
        s = lax.axis_index("s")

        def zfill(i, _):
            zv[i, :] = jnp.zeros((16,), jnp.float32)
            return 0
        lax.fori_loop(0, zr, zfill, 0)

        def window(base_row, mult, off, raw_idx, nwin):
            def win(w, _):
                ro = base_row + w * NCH
                pltpu.sync_copy(src_r.at[pl.ds(ro, NCH)], srcv)
                pltpu.sync_copy(dst_r.at[pl.ds(ro, NCH)], dstv)
                if not raw_idx:
                    def idxbody(j, _):
                        r = j // 8
                        cc = (j % 8) * 16
                        v = srcv[r, pl.ds(cc, 16)]
                        idxv[r, pl.ds(cc, 16)] = v * mult + off
                        return 0
                    lax.fori_loop(0, NCH * 8, idxbody, 0)
                    iref = idxv
                else:
                    iref = srcv
                descs = [
                    pltpu.async_copy(tab_r.at[iref.at[j]],
                                     rowsv.at[pl.ds(j * CH, CH)], sem)
                    for j in range(NCH)
                ]
                for d in descs:
                    d.wait()
                for j in range(NCH):
                    pltpu.sync_copy(rowsv.at[pl.ds(j * CH, CH)],
                                    acc.at[dstv.at[j]], add=True)
                return 0
            lax.fori_loop(0, nwin, win, 0)

        def zero_acc():
            for k in range(32):
                pltpu.sync_copy(zv, acc.at[pl.ds(s * rpt + k * zr, zr)])

        def writeout(obi):
            pltpu.sync_copy(acc.at[pl.ds(s * rpt, rpt)],
                            out_r.at[obi, pl.ds(s * rpt, rpt), :])

        if split:
            mult, off = blocks[0]
            zero_acc()
            plsc.subcore_barrier()
            base = (c * 16 + s) * nw * NCH
            window(base, mult, off, raw_idx=(mult == 1 and off == 0), nwin=nw)
            plsc.subcore_barrier()
            writeout(c)
            plsc.subcore_barrier()
        else:
            for r in range(nb // 2):
                f0, f1 = 2 * r, 2 * r + 1
                mult = jnp.where(c == 0, blocks[f0][0], blocks[f1][0])
                off = jnp.where(c == 0, blocks[f0][1], blocks[f1][1])
                zero_acc()
                plsc.subcore_barrier()
                window(s * nw * NCH, mult, off, raw_idx=False, nwin=nw)
                plsc.subcore_barrier()
                writeout(f0 + c)
                plsc.subcore_barrier()
            if odd:
                mult, off = blocks[nb - 1]
                zero_acc()
                plsc.subcore_barrier()
                base = (c * 16 + s) * nws * NCH
                window(base, mult, off, raw_idx=False, nwin=nws)
                plsc.subcore_barrier()
                writeout(nb - 1 + c)
                plsc.subcore_barrier()

    kern = pl.kernel(
        body,
        out_type=jax.ShapeDtypeStruct((nout, npad, 16), jnp.float32),
        mesh=mesh,
        scratch_types=[
            pltpu.VMEM((NCH, CH), jnp.int32),
            pltpu.VMEM((NCH, CH), jnp.int32),
            pltpu.VMEM((NCH, CH), jnp.int32),
            pltpu.VMEM((WIN, 16), jnp.float32),
            pltpu.VMEM((zr, 16), jnp.float32),
            pltpu.VMEM_SHARED((npad, 16), jnp.float32),
            pltpu.SemaphoreType.DMA,
        ],
        compiler_params=pltpu.CompilerParams(use_tc_tiling_on_sc=False),
    )
    return kern(tab, srcr, dstr)


def _k_pre(xp, bt, w0, npad):
    grid = npad // BN

    def body(x_r, b_r, w_r, p0_r, cg_r):
        i = pl.program_id(0)
        mk = (b_r[...] < B64).astype(jnp.float32)
        p0_r[...] = x_r[...] * w_r[0:1, :]
        oh = _onehot(b_r[...])

        @pl.when(i == 0)
        def _():
            cg_r[...] = jnp.zeros_like(cg_r)
        cg_r[...] += _mmT(oh, mk)

    return pl.pallas_call(
        body,
        grid=(grid,),
        in_specs=[
            pl.BlockSpec((BN, 1), lambda i: (i, 0)),
            pl.BlockSpec((BN, 1), lambda i: (i, 0)),
            pl.BlockSpec((8, 16), lambda i: (0, 0)),
        ],
        out_specs=[
            pl.BlockSpec((BN, 16), lambda i: (i, 0)),
            pl.BlockSpec((B64, 1), lambda i: (0, 0)),
        ],
        out_shape=[
            jax.ShapeDtypeStruct((npad, 16), jnp.float32),
            jax.ShapeDtypeStruct((B64, 1), jnp.float32),
        ],
    )(xp, bt, w0)


def _k_inv(agg0, npad):
    grid = npad // BN

    def body(c_r, inv_r):
        deg = c_r[0][:, 0:1]
        inv_r[...] = 1.0 / jnp.maximum(deg, 1.0)

    return pl.pallas_call(
        body,
        grid=(grid,),
        in_specs=[pl.BlockSpec((1, BN, 16), lambda i: (1, i, 0))],
        out_specs=pl.BlockSpec((BN, 1), lambda i: (i, 0)),
        out_shape=jax.ShapeDtypeStruct((npad, 1), jnp.float32),
    )(agg0)


def _k1(agg, h, inv, bt, wl, wr, cp, mode, wout, npad):
    grid = npad // BN
    a_dim = agg.shape[0]
    hw = h.shape[1]

    def body(a_r, h_r, inv_r, b_r, wl_r, wr_r, cp_r, z_r, s1_r):
        i = pl.program_id(0)
        inv = inv_r[...]
        if mode == "l0":
            acc = a_r[0] * inv
            rterm = h_r[...] * wr_r[0:1, :]
        elif mode == "sum2":
            acc = _mm((a_r[0] + a_r[1]) * inv, wl_r[...])
            rterm = _mm(h_r[...], wr_r[...])
        else:
            acc = _mm(a_r[0] * inv, wl_r[pl.ds(0, 16), :])
            for f in range(1, 6):
                acc += _mm(a_r[f] * inv, wl_r[pl.ds(f * 16, 16), :])
            acc += _mm((a_r[6] + a_r[7]) * inv, wl_r[pl.ds(96, 16), :])
            rterm = _mm(h_r[...], wr_r[...])
        mk = (b_r[...] < B64).astype(jnp.float32)
        z = (acc + rterm + cp_r[3:4, :]) * mk
        z_r[...] = z
        oh = _onehot(b_r[...])

        @pl.when(i == 0)
        def _():
            s1_r[...] = jnp.zeros_like(s1_r)
        s1_r[...] += _mmT(oh, z)

    return pl.pallas_call(
        body,
        grid=(grid,),
        in_specs=[
            pl.BlockSpec((a_dim, BN, 16), lambda i: (0, i, 0)),
            pl.BlockSpec((BN, hw), lambda i: (i, 0)),
            pl.BlockSpec((BN, 1), lambda i: (i, 0)),
            pl.BlockSpec((BN, 1), lambda i: (i, 0)),
            pl.BlockSpec(wl.shape, lambda i: (0, 0)),
            pl.BlockSpec(wr.shape, lambda i: (0, 0)),
            pl.BlockSpec((8, wout), lambda i: (0, 0)),
        ],
        out_specs=[
            pl.BlockSpec((BN, wout), lambda i: (i, 0)),
            pl.BlockSpec((B64, wout), lambda i: (0, 0)),
        ],
        out_shape=[
            jax.ShapeDtypeStruct((npad, wout), jnp.float32),
            jax.ShapeDtypeStruct((B64, wout), jnp.float32),
        ],
    )(agg, h, inv, bt, wl, wr, cp)


def _k2(z, s1, cg, bt, cp, wout, npad):
    grid = npad // BN

    def body(z_r, s1_r, cg_r, b_r, cp_r, s2_r):
        i = pl.program_id(0)
        mean_g = s1_r[...] / jnp.maximum(cg_r[...], 1.0)
        oh = _onehot(b_r[...])
        meanb = _mm(oh, mean_g)
        clone = z_r[...] - cp_r[0:1, :] * meanb

        @pl.when(i == 0)
        def _():
            s2_r[...] = jnp.zeros_like(s2_r)
        s2_r[...] += _mmT(oh, clone * clone)

    return pl.pallas_call(
        body,
        grid=(grid,),
        in_specs=[
            pl.BlockSpec((BN, wout), lambda i: (i, 0)),
            pl.BlockSpec((B64, wout), lambda i: (0, 0)),
            pl.BlockSpec((B64, 1), lambda i: (0, 0)),
            pl.BlockSpec((BN, 1), lambda i: (i, 0)),
            pl.BlockSpec((8, wout), lambda i: (0, 0)),
        ],
        out_specs=pl.BlockSpec((B64, wout), lambda i: (0, 0)),
        out_shape=jax.ShapeDtypeStruct((B64, wout), jnp.float32),
    )(z, s1, cg, bt, cp)


def _k3(z, s1, s2, cg, bt, cp, wout, npad):
    grid = npad // BN

    def body(z_r, s1_r, s2_r, cg_r, b_r, cp_r, h_r):
        cg = cg_r[...]
        mean_g = s1_r[...] / jnp.maximum(cg, 1.0)
        sq = jnp.floor(jnp.sqrt(cg))
        s2 = s2_r[...]
        factor = jnp.where(s2 > 0.0, sq / jnp.sqrt(s2), 0.0)
        oh = _onehot(b_r[...])
        meanb = _mm(oh, mean_g)
        fb = _mm(oh, factor)
        clone = z_r[...] - cp_r[0:1, :] * meanb
        mk = (b_r[...] < B64).astype(jnp.float32)
        h_r[...] = jnp.maximum(clone * fb * cp_r[1:2, :] + cp_r[2:3, :],
                               0.0) * mk

    return pl.pallas_call(
        body,
        grid=(grid,),
        in_specs=[
            pl.BlockSpec((BN, wout), lambda i: (i, 0)),
            pl.BlockSpec((B64, wout), lambda i: (0, 0)),
            pl.BlockSpec((B64, wout), lambda i: (0, 0)),
            pl.BlockSpec((B64, 1), lambda i: (0, 0)),
            pl.BlockSpec((BN, 1), lambda i: (i, 0)),
            pl.BlockSpec((8, wout), lambda i: (0, 0)),
        ],
        out_specs=pl.BlockSpec((BN, wout), lambda i: (i, 0)),
        out_shape=jax.ShapeDtypeStruct((npad, wout), jnp.float32),
    )(z, s1, s2, cg, bt, cp)


def _k_yr(h6, w7, npad):
    grid = npad // BN

    def body(h_r, w_r, y_r, r_r):
        yr = _mm(h_r[...], w_r[...])
        y = yr[:, 0:1]
        y_r[...] = jnp.concatenate(
            [y, jnp.zeros((BN, 15), jnp.float32)], axis=1)
        r_r[...] = yr[:, 1:2]

    return pl.pallas_call(
        body,
        grid=(grid,),
        in_specs=[
            pl.BlockSpec((BN, 112), lambda i: (i, 0)),
            pl.BlockSpec((112, 8), lambda i: (0, 0)),
        ],
        out_specs=[
            pl.BlockSpec((BN, 16), lambda i: (i, 0)),
            pl.BlockSpec((BN, 1), lambda i: (i, 0)),
        ],
        out_shape=[
            jax.ShapeDtypeStruct((npad, 16), jnp.float32),
            jax.ShapeDtypeStruct((npad, 1), jnp.float32),
        ],
    )(h6, w7)


def _k_final(agg7, inv, r, bt, cg, cp7, npad):
    grid = npad // BN

    def body(a_r, inv_r, r_r, b_r, cg_r, cp_r, ret_r):
        i = pl.program_id(0)
        aggy = (a_r[0] + a_r[1])[:, 0:1]
        mk = (b_r[...] < B64).astype(jnp.float32)
        o = (aggy * inv_r[...] + cp_r[3:4, 0:1] + r_r[...]) * mk
        oh = _onehot(b_r[...])

        @pl.when(i == 0)
        def _():
            ret_r[...] = jnp.zeros_like(ret_r)
        ret_r[...] += _mmT(oh, o)

        @pl.when(i == grid - 1)
        def _():
            ret_r[...] = ret_r[...] / jnp.maximum(cg_r[...], 1.0)

    return pl.pallas_call(
        body,
        grid=(grid,),
        in_specs=[
            pl.BlockSpec((2, BN, 16), lambda i: (0, i, 0)),
            pl.BlockSpec((BN, 1), lambda i: (i, 0)),
            pl.BlockSpec((BN, 1), lambda i: (i, 0)),
            pl.BlockSpec((BN, 1), lambda i: (i, 0)),
            pl.BlockSpec((B64, 1), lambda i: (0, 0)),
            pl.BlockSpec((8, 16), lambda i: (0, 0)),
        ],
        out_specs=pl.BlockSpec((B64, 1), lambda i: (0, 0)),
        out_shape=jax.ShapeDtypeStruct((B64, 1), jnp.float32),
    )(agg7, inv, r, bt, cg, cp7)


def _padT(w, rows, cols):
    wt = w.T.astype(jnp.float32)
    return jnp.pad(wt, ((0, rows - wt.shape[0]), (0, cols - wt.shape[1])))


def kernel(x, params, edge_index, batch):
    N = x.shape[0]
    E = edge_index.shape[1]
    npad = ((N + BN - 1) // BN) * BN
    per_tile = ((E + 32 * WIN - 1) // (32 * WIN)) * WIN
    epad = 32 * per_tile
    erows = epad // CH
    pade = epad - E

    src = edge_index[0].astype(jnp.int32)
    dst = edge_index[1].astype(jnp.int32)
    pidx = jnp.arange(pade, dtype=jnp.int32)
    srcr = jnp.concatenate([src, (pidx * 97) % N]).reshape(erows, CH)
    dstr = jnp.concatenate([dst, N + pidx % (npad - N)]).reshape(erows, CH)

    xp = jnp.pad(x.astype(jnp.float32), ((0, npad - N), (0, 0)))
    bt = jnp.pad(batch.astype(jnp.int32), (0, npad - N),
                 constant_values=B64).reshape(npad, 1)
    ones16 = jnp.ones((npad, 16), jnp.float32)

    L = params["layers"]
    Nm = params["norms"]
    wouts = [16] + [112] * 6
    wl = [None] * 8
    wr = [None] * 8
    cp = [None] * 8
    wl[0] = _padT(L[0]["Wl"], 8, 16)
    wr[0] = _padT(L[0]["Wr"], 8, 16)
    wl[1] = _padT(L[1]["Wl"], 16, 112)
    wr[1] = _padT(L[1]["Wr"], 16, 112)
    for i in range(2, 7):
        wl[i] = _padT(L[i]["Wl"], 112, 112)
        wr[i] = _padT(L[i]["Wr"], 112, 112)
    w7 = jnp.concatenate([_padT(L[7]["Wl"], 112, 1),
                          _padT(L[7]["Wr"], 112, 1),
                          jnp.zeros((112, 6), jnp.float32)], axis=1)
    for i in range(7):
        w = wouts[i]
        cp[i] = jnp.stack([
            jnp.pad(Nm[i]["alpha"].astype(jnp.float32), (0, w - Nm[i]["alpha"].shape[0])),
            jnp.pad(Nm[i]["scale"].astype(jnp.float32), (0, w - Nm[i]["scale"].shape[0])),
            jnp.pad(Nm[i]["shift"].astype(jnp.float32), (0, w - Nm[i]["shift"].shape[0])),
            jnp.pad(L[i]["bl"].astype(jnp.float32), (0, w - L[i]["bl"].shape[0])),
        ] + [jnp.zeros((w,), jnp.float32)] * 4)
    cp7 = jnp.zeros((8, 16), jnp.float32).at[3, 0].set(L[7]["bl"][0])

    p0, cg = _k_pre(xp, bt, wl[0], npad)
    tab0 = jnp.concatenate([p0, ones16], axis=0)
    agg0 = _sc_agg(tab0, srcr, dstr, ((1, 0), (1, npad)), False, npad)
    inv = _k_inv(agg0, npad)
    z, s1 = _k1(agg0, xp, inv, bt, wl[0], wr[0], cp[0], "l0", 16, npad)
    s2 = _k2(z, s1, cg, bt, cp[0], 16, npad)
    h = _k3(z, s1, s2, cg, bt, cp[0], 16, npad)

    agg = _sc_agg(h, srcr, dstr, ((1, 0),), True, npad)
    z, s1 = _k1(agg, h, inv, bt, wl[1], wr[1], cp[1], "sum2", 112, npad)
    s2 = _k2(z, s1, cg, bt, cp[1], 112, npad)
    h = _k3(z, s1, s2, cg, bt, cp[1], 112, npad)

    blocks7 = tuple((7, f) for f in range(7))
    for i in range(2, 7):
        hv = h.reshape(npad * 7, 16)
        agg = _sc_agg(hv, srcr, dstr, blocks7, False, npad)
        z, s1 = _k1(agg, h, inv, bt, wl[i], wr[i], cp[i], "cat7", 112, npad)
        s2 = _k2(z, s1, cg, bt, cp[i], 112, npad)
        h = _k3(z, s1, s2, cg, bt, cp[i], 112, npad)

    y16, r = _k_yr(h, w7, npad)
    agg7 = _sc_agg(y16, srcr, dstr, ((1, 0),), True, npad)
    ret = _k_final(agg7, inv, r, bt, cg, cp7, npad)
    return ret.reshape(B64)


# speedup vs baseline: 5.4825x; 1.0153x over previous
import functools

import jax
import jax.numpy as jnp
from jax import lax
from jax.experimental import pallas as pl
from jax.experimental.pallas import tpu as pltpu
from jax.experimental.pallas import tpu_sc as plsc

B64 = 64
BN = 2048
WIN = 512
CH = 128
NCH = WIN // CH

_HIGH = jax.lax.Precision.HIGHEST


def _mm(a, b):
    return jax.lax.dot_general(a, b, (((1,), (0,)), ((), ())),
                               precision=_HIGH,
                               preferred_element_type=jnp.float32)


def _mmT(a, b):
    return jax.lax.dot_general(a, b, (((0,), (0,)), ((), ())),
                               precision=_HIGH,
                               preferred_element_type=jnp.float32)


def _onehot(batch_blk):
    g = jax.lax.broadcasted_iota(jnp.int32, (1, B64), 1)
    return (batch_blk == g).astype(jnp.float32)


def _sc_agg(tab, srcr, dstr, blocks, split, npad):
    erows = srcr.shape[0]
    nb = len(blocks)
    odd = (not split) and (nb % 2 == 1) and nb > 1
    nout = 2 if split else nb + (1 if odd else 0)
    rpt = npad // 16
    zr = rpt // 32
    if split:
        nw = erows // (32 * NCH)
    else:
        nw = erows // (16 * NCH)
        nws = nw // 2

    mesh = plsc.VectorSubcoreMesh(core_axis_name="c", subcore_axis_name="s",
                                  num_cores=2, num_subcores=16)

    def body(tab_r, src_r, dst_r, out_r, srcv, dstv, idxv, rowsv,
             srcv2, dstv2, idxv2, rowsv2, zv, acc, sem, sem2):
        c = lax.axis_index("c")
        s = lax.axis_index("s")

        def zfill(i, _):
            zv[i, :] = jnp.zeros((16,), jnp.float32)
            return 0
        lax.fori_loop(0, zr, zfill, 0)

        def window(base_row, mult, off, raw_idx, nwin):
            def prep(ro, srcv_, dstv_, idxv_):
                pltpu.sync_copy(src_r.at[pl.ds(ro, NCH)], srcv_)
                pltpu.sync_copy(dst_r.at[pl.ds(ro, NCH)], dstv_)
                if not raw_idx:
                    def idxbody(j, _):
                        r = j // 8
                        cc = (j % 8) * 16
                        v = srcv_[r, pl.ds(cc, 16)]
                        idxv_[r, pl.ds(cc, 16)] = v * mult + off
                        return 0
                    lax.fori_loop(0, NCH * 8, idxbody, 0)
                    return idxv_
                return srcv_

            def gath(iref, rowsv_, sem_):
                return [
                    pltpu.async_copy(tab_r.at[iref.at[j]],
                                     rowsv_.at[pl.ds(j * CH, CH)], sem_)
                    for j in range(NCH)
                ]

            def scat(rowsv_, dstv_):
                for j in range(NCH):
                    pltpu.sync_copy(rowsv_.at[pl.ds(j * CH, CH)],
                                    acc.at[dstv_.at[j]], add=True)

            def win(t, _):
                ro = base_row + 2 * t * NCH
                da = gath(prep(ro, srcv, dstv, idxv), rowsv, sem)
                db = gath(prep(ro + NCH, srcv2, dstv2, idxv2), rowsv2, sem2)
                for d in da:
                    d.wait()
                scat(rowsv, dstv)
                for d in db:
                    d.wait()
                scat(rowsv2, dstv2)
                return 0
            lax.fori_loop(0, nwin // 2, win, 0)

        def zero_acc():
            for k in range(32):
                pltpu.sync_copy(zv, acc.at[pl.ds(s * rpt + k * zr, zr)])

        def writeout(obi):
            pltpu.sync_copy(acc.at[pl.ds(s * rpt, rpt)],
                            out_r.at[obi, pl.ds(s * rpt, rpt), :])

        if split:
            mult, off = blocks[0]
            zero_acc()
            plsc.subcore_barrier()
            base = (c * 16 + s) * nw * NCH
            window(base, mult, off, raw_idx=(mult == 1 and off == 0), nwin=nw)
            plsc.subcore_barrier()
            writeout(c)
            plsc.subcore_barrier()
        else:
            for r in range(nb // 2):
                f0, f1 = 2 * r, 2 * r + 1
                mult = jnp.where(c == 0, blocks[f0][0], blocks[f1][0])
                off = jnp.where(c == 0, blocks[f0][1], blocks[f1][1])
                zero_acc()
                plsc.subcore_barrier()
                window(s * nw * NCH, mult, off, raw_idx=False, nwin=nw)
                plsc.subcore_barrier()
                writeout(f0 + c)
                plsc.subcore_barrier()
            if odd:
                mult, off = blocks[nb - 1]
                zero_acc()
                plsc.subcore_barrier()
                base = (c * 16 + s) * nws * NCH
                window(base, mult, off, raw_idx=False, nwin=nws)
                plsc.subcore_barrier()
                writeout(nb - 1 + c)
                plsc.subcore_barrier()

    kern = pl.kernel(
        body,
        out_type=jax.ShapeDtypeStruct((nout, npad, 16), jnp.float32),
        mesh=mesh,
        scratch_types=[
            pltpu.VMEM((NCH, CH), jnp.int32),
            pltpu.VMEM((NCH, CH), jnp.int32),
            pltpu.VMEM((NCH, CH), jnp.int32),
            pltpu.VMEM((WIN, 16), jnp.float32),
            pltpu.VMEM((NCH, CH), jnp.int32),
            pltpu.VMEM((NCH, CH), jnp.int32),
            pltpu.VMEM((NCH, CH), jnp.int32),
            pltpu.VMEM((WIN, 16), jnp.float32),
            pltpu.VMEM((zr, 16), jnp.float32),
            pltpu.VMEM_SHARED((npad, 16), jnp.float32),
            pltpu.SemaphoreType.DMA,
            pltpu.SemaphoreType.DMA,
        ],
        compiler_params=pltpu.CompilerParams(use_tc_tiling_on_sc=False),
    )
    return kern(tab, srcr, dstr)


def _k_pre(xp, bt, w0, npad):
    grid = npad // BN

    def body(x_r, b_r, w_r, p0_r, cg_r):
        i = pl.program_id(0)
        mk = (b_r[...] < B64).astype(jnp.float32)
        p0_r[...] = x_r[...] * w_r[0:1, :]
        oh = _onehot(b_r[...])

        @pl.when(i == 0)
        def _():
            cg_r[...] = jnp.zeros_like(cg_r)
        cg_r[...] += _mmT(oh, mk)

    return pl.pallas_call(
        body,
        grid=(grid,),
        in_specs=[
            pl.BlockSpec((BN, 1), lambda i: (i, 0)),
            pl.BlockSpec((BN, 1), lambda i: (i, 0)),
            pl.BlockSpec((8, 16), lambda i: (0, 0)),
        ],
        out_specs=[
            pl.BlockSpec((BN, 16), lambda i: (i, 0)),
            pl.BlockSpec((B64, 1), lambda i: (0, 0)),
        ],
        out_shape=[
            jax.ShapeDtypeStruct((npad, 16), jnp.float32),
            jax.ShapeDtypeStruct((B64, 1), jnp.float32),
        ],
    )(xp, bt, w0)


def _k_inv(agg0, npad):
    grid = npad // BN

    def body(c_r, inv_r):
        deg = c_r[0][:, 0:1]
        inv_r[...] = 1.0 / jnp.maximum(deg, 1.0)

    return pl.pallas_call(
        body,
        grid=(grid,),
        in_specs=[pl.BlockSpec((1, BN, 16), lambda i: (1, i, 0))],
        out_specs=pl.BlockSpec((BN, 1), lambda i: (i, 0)),
        out_shape=jax.ShapeDtypeStruct((npad, 1), jnp.float32),
    )(agg0)


def _k1(agg, h, inv, bt, wl, wr, cp, mode, wout, npad):
    grid = npad // BN
    a_dim = agg.shape[0]
    hw = h.shape[1]

    def body(a_r, h_r, inv_r, b_r, wl_r, wr_r, cp_r, z_r, s1_r):
        i = pl.program_id(0)
        inv = inv_r[...]
        if mode == "l0":
            acc = a_r[0] * inv
            rterm = h_r[...] * wr_r[0:1, :]
        elif mode == "sum2":
            acc = _mm((a_r[0] + a_r[1]) * inv, wl_r[...])
            rterm = _mm(h_r[...], wr_r[...])
        else:
            acc = _mm(a_r[0] * inv, wl_r[pl.ds(0, 16), :])
            for f in range(1, 6):
                acc += _mm(a_r[f] * inv, wl_r[pl.ds(f * 16, 16), :])
            acc += _mm((a_r[6] + a_r[7]) * inv, wl_r[pl.ds(96, 16), :])
            rterm = _mm(h_r[...], wr_r[...])
        mk = (b_r[...] < B64).astype(jnp.float32)
        z = (acc + rterm + cp_r[3:4, :]) * mk
        z_r[...] = z
        oh = _onehot(b_r[...])

        @pl.when(i == 0)
        def _():
            s1_r[...] = jnp.zeros_like(s1_r)
        s1_r[...] += _mmT(oh, z)

    return pl.pallas_call(
        body,
        grid=(grid,),
        in_specs=[
            pl.BlockSpec((a_dim, BN, 16), lambda i: (0, i, 0)),
            pl.BlockSpec((BN, hw), lambda i: (i, 0)),
            pl.BlockSpec((BN, 1), lambda i: (i, 0)),
            pl.BlockSpec((BN, 1), lambda i: (i, 0)),
            pl.BlockSpec(wl.shape, lambda i: (0, 0)),
            pl.BlockSpec(wr.shape, lambda i: (0, 0)),
            pl.BlockSpec((8, wout), lambda i: (0, 0)),
        ],
        out_specs=[
            pl.BlockSpec((BN, wout), lambda i: (i, 0)),
            pl.BlockSpec((B64, wout), lambda i: (0, 0)),
        ],
        out_shape=[
            jax.ShapeDtypeStruct((npad, wout), jnp.float32),
            jax.ShapeDtypeStruct((B64, wout), jnp.float32),
        ],
    )(agg, h, inv, bt, wl, wr, cp)


def _k2(z, s1, cg, bt, cp, wout, npad):
    grid = npad // BN

    def body(z_r, s1_r, cg_r, b_r, cp_r, s2_r):
        i = pl.program_id(0)
        mean_g = s1_r[...] / jnp.maximum(cg_r[...], 1.0)
        oh = _onehot(b_r[...])
        meanb = _mm(oh, mean_g)
        clone = z_r[...] - cp_r[0:1, :] * meanb

        @pl.when(i == 0)
        def _():
            s2_r[...] = jnp.zeros_like(s2_r)
        s2_r[...] += _mmT(oh, clone * clone)

    return pl.pallas_call(
        body,
        grid=(grid,),
        in_specs=[
            pl.BlockSpec((BN, wout), lambda i: (i, 0)),
            pl.BlockSpec((B64, wout), lambda i: (0, 0)),
            pl.BlockSpec((B64, 1), lambda i: (0, 0)),
            pl.BlockSpec((BN, 1), lambda i: (i, 0)),
            pl.BlockSpec((8, wout), lambda i: (0, 0)),
        ],
        out_specs=pl.BlockSpec((B64, wout), lambda i: (0, 0)),
        out_shape=jax.ShapeDtypeStruct((B64, wout), jnp.float32),
    )(z, s1, cg, bt, cp)


def _k3(z, s1, s2, cg, bt, cp, wout, npad):
    grid = npad // BN

    def body(z_r, s1_r, s2_r, cg_r, b_r, cp_r, h_r):
        cg = cg_r[...]
        mean_g = s1_r[...] / jnp.maximum(cg, 1.0)
        sq = jnp.floor(jnp.sqrt(cg))
        s2 = s2_r[...]
        factor = jnp.where(s2 > 0.0, sq / jnp.sqrt(s2), 0.0)
        oh = _onehot(b_r[...])
        meanb = _mm(oh, mean_g)
        fb = _mm(oh, factor)
        clone = z_r[...] - cp_r[0:1, :] * meanb
        mk = (b_r[...] < B64).astype(jnp.float32)
        h_r[...] = jnp.maximum(clone * fb * cp_r[1:2, :] + cp_r[2:3, :],
                               0.0) * mk

    return pl.pallas_call(
        body,
        grid=(grid,),
        in_specs=[
            pl.BlockSpec((BN, wout), lambda i: (i, 0)),
            pl.BlockSpec((B64, wout), lambda i: (0, 0)),
            pl.BlockSpec((B64, wout), lambda i: (0, 0)),
            pl.BlockSpec((B64, 1), lambda i: (0, 0)),
            pl.BlockSpec((BN, 1), lambda i: (i, 0)),
            pl.BlockSpec((8, wout), lambda i: (0, 0)),
        ],
        out_specs=pl.BlockSpec((BN, wout), lambda i: (i, 0)),
        out_shape=jax.ShapeDtypeStruct((npad, wout), jnp.float32),
    )(z, s1, s2, cg, bt, cp)


def _k_yr(h6, w7, npad):
    grid = npad // BN

    def body(h_r, w_r, y_r, r_r):
        yr = _mm(h_r[...], w_r[...])
        y = yr[:, 0:1]
        y_r[...] = jnp.concatenate(
            [y, jnp.zeros((BN, 15), jnp.float32)], axis=1)
        r_r[...] = yr[:, 1:2]

    return pl.pallas_call(
        body,
        grid=(grid,),
        in_specs=[
            pl.BlockSpec((BN, 112), lambda i: (i, 0)),
            pl.BlockSpec((112, 8), lambda i: (0, 0)),
        ],
        out_specs=[
            pl.BlockSpec((BN, 16), lambda i: (i, 0)),
            pl.BlockSpec((BN, 1), lambda i: (i, 0)),
        ],
        out_shape=[
            jax.ShapeDtypeStruct((npad, 16), jnp.float32),
            jax.ShapeDtypeStruct((npad, 1), jnp.float32),
        ],
    )(h6, w7)


def _k_final(agg7, inv, r, bt, cg, cp7, npad):
    grid = npad // BN

    def body(a_r, inv_r, r_r, b_r, cg_r, cp_r, ret_r):
        i = pl.program_id(0)
        aggy = (a_r[0] + a_r[1])[:, 0:1]
        mk = (b_r[...] < B64).astype(jnp.float32)
        o = (aggy * inv_r[...] + cp_r[3:4, 0:1] + r_r[...]) * mk
        oh = _onehot(b_r[...])

        @pl.when(i == 0)
        def _():
            ret_r[...] = jnp.zeros_like(ret_r)
        ret_r[...] += _mmT(oh, o)

        @pl.when(i == grid - 1)
        def _():
            ret_r[...] = ret_r[...] / jnp.maximum(cg_r[...], 1.0)

    return pl.pallas_call(
        body,
        grid=(grid,),
        in_specs=[
            pl.BlockSpec((2, BN, 16), lambda i: (0, i, 0)),
            pl.BlockSpec((BN, 1), lambda i: (i, 0)),
            pl.BlockSpec((BN, 1), lambda i: (i, 0)),
            pl.BlockSpec((BN, 1), lambda i: (i, 0)),
            pl.BlockSpec((B64, 1), lambda i: (0, 0)),
            pl.BlockSpec((8, 16), lambda i: (0, 0)),
        ],
        out_specs=pl.BlockSpec((B64, 1), lambda i: (0, 0)),
        out_shape=jax.ShapeDtypeStruct((B64, 1), jnp.float32),
    )(agg7, inv, r, bt, cg, cp7)


def _padT(w, rows, cols):
    wt = w.T.astype(jnp.float32)
    return jnp.pad(wt, ((0, rows - wt.shape[0]), (0, cols - wt.shape[1])))


def kernel(x, params, edge_index, batch):
    N = x.shape[0]
    E = edge_index.shape[1]
    npad = ((N + BN - 1) // BN) * BN
    per_tile = ((E + 64 * WIN - 1) // (64 * WIN)) * (2 * WIN)
    epad = 32 * per_tile
    erows = epad // CH
    pade = epad - E

    src = edge_index[0].astype(jnp.int32)
    dst = edge_index[1].astype(jnp.int32)
    pidx = jnp.arange(pade, dtype=jnp.int32)
    srcr = jnp.concatenate([src, (pidx * 97) % N]).reshape(erows, CH)
    dstr = jnp.concatenate([dst, N + pidx % (npad - N)]).reshape(erows, CH)

    xp = jnp.pad(x.astype(jnp.float32), ((0, npad - N), (0, 0)))
    bt = jnp.pad(batch.astype(jnp.int32), (0, npad - N),
                 constant_values=B64).reshape(npad, 1)
    ones16 = jnp.ones((npad, 16), jnp.float32)

    L = params["layers"]
    Nm = params["norms"]
    wouts = [16] + [112] * 6
    wl = [None] * 8
    wr = [None] * 8
    cp = [None] * 8
    wl[0] = _padT(L[0]["Wl"], 8, 16)
    wr[0] = _padT(L[0]["Wr"], 8, 16)
    wl[1] = _padT(L[1]["Wl"], 16, 112)
    wr[1] = _padT(L[1]["Wr"], 16, 112)
    for i in range(2, 7):
        wl[i] = _padT(L[i]["Wl"], 112, 112)
        wr[i] = _padT(L[i]["Wr"], 112, 112)
    w7 = jnp.concatenate([_padT(L[7]["Wl"], 112, 1),
                          _padT(L[7]["Wr"], 112, 1),
                          jnp.zeros((112, 6), jnp.float32)], axis=1)
    for i in range(7):
        w = wouts[i]
        cp[i] = jnp.stack([
            jnp.pad(Nm[i]["alpha"].astype(jnp.float32), (0, w - Nm[i]["alpha"].shape[0])),
            jnp.pad(Nm[i]["scale"].astype(jnp.float32), (0, w - Nm[i]["scale"].shape[0])),
            jnp.pad(Nm[i]["shift"].astype(jnp.float32), (0, w - Nm[i]["shift"].shape[0])),
            jnp.pad(L[i]["bl"].astype(jnp.float32), (0, w - L[i]["bl"].shape[0])),
        ] + [jnp.zeros((w,), jnp.float32)] * 4)
    cp7 = jnp.zeros((8, 16), jnp.float32).at[3, 0].set(L[7]["bl"][0])

    p0, cg = _k_pre(xp, bt, wl[0], npad)
    tab0 = jnp.concatenate([p0, ones16], axis=0)
    agg0 = _sc_agg(tab0, srcr, dstr, ((1, 0), (1, npad)), False, npad)
    inv = _k_inv(agg0, npad)
    z, s1 = _k1(agg0, xp, inv, bt, wl[0], wr[0], cp[0], "l0", 16, npad)
    s2 = _k2(z, s1, cg, bt, cp[0], 16, npad)
    h = _k3(z, s1, s2, cg, bt, cp[0], 16, npad)

    agg = _sc_agg(h, srcr, dstr, ((1, 0),), True, npad)
    z, s1 = _k1(agg, h, inv, bt, wl[1], wr[1], cp[1], "sum2", 112, npad)
    s2 = _k2(z, s1, cg, bt, cp[1], 112, npad)
    h = _k3(z, s1, s2, cg, bt, cp[1], 112, npad)

    blocks7 = tuple((7, f) for f in range(7))
    for i in range(2, 7):
        hv = h.reshape(npad * 7, 16)
        agg = _sc_agg(hv, srcr, dstr, blocks7, False, npad)
        z, s1 = _k1(agg, h, inv, bt, wl[i], wr[i], cp[i], "cat7", 112, npad)
        s2 = _k2(z, s1, cg, bt, cp[i], 112, npad)
        h = _k3(z, s1, s2, cg, bt, cp[i], 112, npad)

    y16, r = _k_yr(h, w7, npad)
    agg7 = _sc_agg(y16, srcr, dstr, ((1, 0),), True, npad)
    ret = _k_final(agg7, inv, r, bt, cg, cp7, npad)
    return ret.reshape(B64)
